# Initial kernel scaffold; baseline (speedup 1.0000x reference)
import functools

import jax
import jax.numpy as jnp
from jax import lax
from jax.experimental import pallas as pl
from jax.experimental.pallas import tpu as pltpu
from jax.experimental.pallas import tpu_sc as plsc

N = 10000
E = 160000
IN_FEATS = 256
N_HIDDEN = 512
N_CLASSES = 256

NC = 2
NS = 16
NW = NC * NS
EPW = E // NW
CHUNK = 128
NCH = -(-EPW // CHUNK)
EPW_PAD = NCH * CHUNK
PADE = EPW_PAD - EPW
NPAD = 10016
ZSTRIPE = NPAD // NS
WSTRIPE = N // NS
F = 128
BN = 1000


def _sc_mesh():
    return plsc.VectorSubcoreMesh(core_axis_name="c", subcore_axis_name="s")


def _deg_body(dstidx, degp, dst_v, ones_v, zbuf, acc):
    core = lax.axis_index("c")
    sub = lax.axis_index("s")
    wid = sub * NC + core
    pltpu.sync_copy(dstidx.at[wid], dst_v)

    def _fill(i, _):
        ones_v[pl.ds(i * 16, 16), 0] = jnp.full((16,), 1.0, jnp.float32)
        zbuf[pl.ds(i * 16, 16), 0] = jnp.zeros((16,), jnp.float32)
        return _

    lax.fori_loop(0, 40, _fill, None)
    pltpu.sync_copy(zbuf.at[pl.ds(0, ZSTRIPE)], acc.at[pl.ds(sub * ZSTRIPE, ZSTRIPE)])
    plsc.subcore_barrier()

    def _scat(j, _):
        pltpu.sync_copy(ones_v.at[pl.ds(0, CHUNK)], acc.at[dst_v.at[j]], add=True)
        return _

    lax.fori_loop(0, NCH, _scat, None)
    plsc.subcore_barrier()
    for k in range(NC):
        @pl.when(core == k)
        def _(k=k):
            pltpu.sync_copy(acc.at[pl.ds(sub * ZSTRIPE, ZSTRIPE)],
                            degp.at[k, pl.ds(sub * ZSTRIPE, ZSTRIPE)])


def _deg_kernel(dstidx):
    return pl.kernel(
        _deg_body,
        out_type=jax.ShapeDtypeStruct((NC, NPAD, 1), jnp.float32),
        mesh=_sc_mesh(),
        scratch_types=[
            pltpu.VMEM((NCH, CHUNK), jnp.int32),
            pltpu.VMEM((640, 1), jnp.float32),
            pltpu.VMEM((640, 1), jnp.float32),
            pltpu.VMEM_SHARED((NPAD, 1), jnp.float32),
        ],
    )(dstidx)


def _prop_body(C, g2, srcidx, dstidx, out, src_v, dst_v, buf, zbuf, acc, sem):
    core = lax.axis_index("c")
    sub = lax.axis_index("s")
    wid = sub * NC + core
    pltpu.sync_copy(srcidx.at[wid], src_v)
    pltpu.sync_copy(dstidx.at[wid], dst_v)

    def _zb(i, _):
        zbuf[i // 8, pl.ds((i % 8) * 16, 16)] = jnp.zeros((16,), jnp.float32)
        return _

    lax.fori_loop(0, 313 * 8, _zb, None)

    for cc in range(C):
        pltpu.sync_copy(zbuf, acc.at[pl.ds(sub * ZSTRIPE, 313)])
        pltpu.sync_copy(zbuf, acc.at[pl.ds(sub * ZSTRIPE + 313, 313)])
        plsc.subcore_barrier()

        def _edge(j, _):
            pltpu.async_copy(g2.at[src_v.at[cc, j]], buf, sem).wait()
            pltpu.sync_copy(buf, acc.at[dst_v.at[j]], add=True)
            return _

        lax.fori_loop(0, NCH, _edge, None)
        plsc.subcore_barrier()
        for k in range(NC):
            @pl.when(core == k)
            def _(k=k, cc=cc):
                pltpu.sync_copy(
                    acc.at[pl.ds(sub * WSTRIPE, WSTRIPE)],
                    out.at[k, pl.ds(cc * N + sub * WSTRIPE, WSTRIPE)])
        plsc.subcore_barrier()


def _prop(C, g2, srcidx, dstidx):
    return pl.kernel(
        functools.partial(_prop_body, C),
        out_type=jax.ShapeDtypeStruct((NC, C * N, F), jnp.float32),
        mesh=_sc_mesh(),
        scratch_types=[
            pltpu.VMEM((C, NCH, CHUNK), jnp.int32),
            pltpu.VMEM((NCH, CHUNK), jnp.int32),
            pltpu.VMEM((CHUNK, F), jnp.float32),
            pltpu.VMEM((313, F), jnp.float32),
            pltpu.VMEM_SHARED((NPAD, F), jnp.float32),
            pltpu.SemaphoreType.DMA,
        ],
    )(g2, srcidx, dstidx)


def _ta_body(degp_ref, x_ref, g_ref, norm_ref):
    deg = degp_ref[0] + degp_ref[1]
    nrm = lax.rsqrt(jnp.maximum(deg, 1.0))
    norm_ref[...] = nrm
    s = x_ref[...] * nrm
    for c in range(IN_FEATS // F):
        g_ref[c] = s[:, c * F:(c + 1) * F]


def _tc_prescale(degp, features):
    CI = IN_FEATS // F
    return pl.pallas_call(
        _ta_body,
        grid=(N // BN,),
        in_specs=[
            pl.BlockSpec((NC, BN, 1), lambda i: (0, i, 0)),
            pl.BlockSpec((BN, IN_FEATS), lambda i: (i, 0)),
        ],
        out_specs=[
            pl.BlockSpec((CI, BN, F), lambda i: (0, i, 0)),
            pl.BlockSpec((BN, 1), lambda i: (i, 0)),
        ],
        out_shape=[
            jax.ShapeDtypeStruct((CI, N, F), jnp.float32),
            jax.ShapeDtypeStruct((N, 1), jnp.float32),
        ],
    )(degp[:, :N], features)


def _tb_body(p_ref, norm_ref, m_ref):
    nrm = norm_ref[...]
    m_ref[0] = (p_ref[0, 0] + p_ref[1, 0]) * (nrm * nrm)


def _tc_mid(p, norm, C):
    p4 = p.reshape(NC, C, N, F)
    return pl.pallas_call(
        _tb_body,
        grid=(C, N // BN),
        in_specs=[
            pl.BlockSpec((NC, 1, BN, F), lambda c, i: (0, c, i, 0)),
            pl.BlockSpec((BN, 1), lambda c, i: (i, 0)),
        ],
        out_specs=pl.BlockSpec((1, BN, F), lambda c, i: (c, i, 0)),
        out_shape=jax.ShapeDtypeStruct((C, N, F), jnp.float32),
    )(p4, norm)


def _tc_layer_body(CI, CO, p_ref, norm_ref, w_ref, g_ref):
    nrm = norm_ref[...]
    acc = jnp.zeros((BN, w_ref.shape[0]), jnp.float32)
    for c in range(CI):
        t = (p_ref[0, c] + p_ref[1, c]) * nrm
        acc = acc + lax.dot_general(
            t, w_ref[:, c * F:(c + 1) * F],
            (((1,), (1,)), ((), ())), preferred_element_type=jnp.float32)
    h = jnp.maximum(acc, 0.0) * nrm
    for co in range(CO):
        g_ref[co] = h[:, co * F:(co + 1) * F]


def _tc_layer(p, norm, W, CI, CO):
    p4 = p.reshape(NC, CI, N, F)
    return pl.pallas_call(
        functools.partial(_tc_layer_body, CI, CO),
        grid=(N // BN,),
        in_specs=[
            pl.BlockSpec((NC, CI, BN, F), lambda i: (0, 0, i, 0)),
            pl.BlockSpec((BN, 1), lambda i: (i, 0)),
            pl.BlockSpec(W.shape, lambda i: (0, 0)),
        ],
        out_specs=pl.BlockSpec((CO, BN, F), lambda i: (0, i, 0)),
        out_shape=jax.ShapeDtypeStruct((CO, N, F), jnp.float32),
    )(p4, norm, W)


def _tc_layer2_body(CI, CO, p_ref, norm_ref, w2_ref, w3_ref, g_ref):
    nrm = norm_ref[...]
    acc = jnp.zeros((BN, N_HIDDEN), jnp.float32)
    for c in range(CI):
        t = (p_ref[0, c] + p_ref[1, c]) * nrm
        acc = acc + lax.dot_general(
            t, w2_ref[:, c * F:(c + 1) * F],
            (((1,), (1,)), ((), ())), preferred_element_type=jnp.float32)
    h = jnp.maximum(acc, 0.0)
    z = lax.dot_general(h, w3_ref[...], (((1,), (1,)), ((), ())),
                        preferred_element_type=jnp.float32)
    g = z * nrm
    for co in range(CO):
        g_ref[co] = g[:, co * F:(co + 1) * F]


def _tc_layer2(p, norm, W2, W3, CI, CO):
    p4 = p.reshape(NC, CI, N, F)
    return pl.pallas_call(
        functools.partial(_tc_layer2_body, CI, CO),
        grid=(N // BN,),
        in_specs=[
            pl.BlockSpec((NC, CI, BN, F), lambda i: (0, 0, i, 0)),
            pl.BlockSpec((BN, 1), lambda i: (i, 0)),
            pl.BlockSpec(W2.shape, lambda i: (0, 0)),
            pl.BlockSpec(W3.shape, lambda i: (0, 0)),
        ],
        out_specs=pl.BlockSpec((CO, BN, F), lambda i: (0, i, 0)),
        out_shape=jax.ShapeDtypeStruct((CO, N, F), jnp.float32),
    )(p4, norm, W2, W3)


def _td_body(p_ref, norm_ref, o_ref):
    nrm = norm_ref[...]
    cols = [(p_ref[0, c] + p_ref[1, c]) * nrm for c in range(N_CLASSES // F)]
    o_ref[...] = jnp.concatenate(cols, axis=1)


def _tc_final(p, norm):
    CI = N_CLASSES // F
    p4 = p.reshape(NC, CI, N, F)
    return pl.pallas_call(
        _td_body,
        grid=(N // BN,),
        in_specs=[
            pl.BlockSpec((NC, CI, BN, F), lambda i: (0, 0, i, 0)),
            pl.BlockSpec((BN, 1), lambda i: (i, 0)),
        ],
        out_specs=pl.BlockSpec((BN, N_CLASSES), lambda i: (i, 0)),
        out_shape=jax.ShapeDtypeStruct((N, N_CLASSES), jnp.float32),
    )(p4, norm)


def kernel(features, edge_index, W1, W2, W3):
    src = edge_index[0]
    dst = edge_index[1]

    w = jnp.arange(NW, dtype=jnp.int32)[:, None]
    i = jnp.arange(PADE, dtype=jnp.int32)[None, :]
    pad_src = (w * 997 + i * 131) % N
    pad_dst = N + (i % 8) + jnp.zeros((NW, 1), jnp.int32)
    srcp = jnp.concatenate([src.reshape(NW, EPW), pad_src], axis=1)
    dstp = jnp.concatenate([dst.reshape(NW, EPW), pad_dst], axis=1)
    dsti = dstp.reshape(NW, NCH, CHUNK)

    def srci(C):
        off = jnp.arange(C, dtype=jnp.int32)[None, :, None] * N
        return (srcp[:, None, :] + off).reshape(NW, C, NCH, CHUNK)

    srci2, srci4 = srci(2), srci(4)

    degp = _deg_kernel(dsti)

    g, norm = _tc_prescale(degp, features)
    p = _prop(2, g.reshape(2 * N, F), srci2, dsti)
    m = _tc_mid(p, norm, 2)
    p = _prop(2, m.reshape(2 * N, F), srci2, dsti)
    g = _tc_layer(p, norm, W1, 2, 4)
    p = _prop(4, g.reshape(4 * N, F), srci4, dsti)
    m = _tc_mid(p, norm, 4)
    p = _prop(4, m.reshape(4 * N, F), srci4, dsti)
    g = _tc_layer2(p, norm, W2, W3, 4, 2)
    p = _prop(2, g.reshape(2 * N, F), srci2, dsti)
    m = _tc_mid(p, norm, 2)
    p = _prop(2, m.reshape(2 * N, F), srci2, dsti)
    return _tc_final(p, norm)


# speedup vs baseline: 3.4235x; 3.4235x over previous
import functools

import jax
import jax.numpy as jnp
from jax import lax
from jax.experimental import pallas as pl
from jax.experimental.pallas import tpu as pltpu
from jax.experimental.pallas import tpu_sc as plsc

N = 10000
E = 160000
IN_FEATS = 256
N_HIDDEN = 512
N_CLASSES = 256

NC = 2
NS = 16
NW = NC * NS
EPW = E // NW
CHUNK = 128
NCH = -(-EPW // CHUNK)
EPW_PAD = NCH * CHUNK
PADE = EPW_PAD - EPW
NPAD = 10240
STRIPE = NPAD // NS
F = 64
BN = 1000


def _sc_mesh():
    return plsc.VectorSubcoreMesh(core_axis_name="c", subcore_axis_name="s")


NDEG = 10240


def _deg_body(dstidx, degp, dst_v, ones_v, zbuf, acc):
    core = lax.axis_index("c")
    sub = lax.axis_index("s")
    wid = sub * NC + core
    pltpu.sync_copy(dstidx.at[wid], dst_v)

    def _fill(i, _):
        ones_v[pl.ds(i * 16, 16)] = jnp.full((16,), 1.0, jnp.float32)
        return _

    def _zero(i, _):
        zbuf[pl.ds(i * 16, 16)] = jnp.zeros((16,), jnp.float32)
        return _

    lax.fori_loop(0, CHUNK // 16, _fill, None)
    lax.fori_loop(0, (NDEG // NS) // 16, _zero, None)
    pltpu.sync_copy(zbuf, acc.at[pl.ds(sub * (NDEG // NS), NDEG // NS)])
    plsc.subcore_barrier()

    def _scat(j, _):
        pltpu.sync_copy(ones_v, acc.at[dst_v.at[j]], add=True)
        return _

    lax.fori_loop(0, NCH, _scat, None)
    plsc.subcore_barrier()
    for k in range(NC):
        @pl.when(core == k)
        def _(k=k):
            pltpu.sync_copy(acc.at[pl.ds(sub * (NDEG // NS), NDEG // NS)],
                            degp.at[k, pl.ds(sub * (NDEG // NS), NDEG // NS)])


def _deg_kernel(dstidx):
    return pl.kernel(
        _deg_body,
        out_type=jax.ShapeDtypeStruct((NC, NDEG), jnp.float32),
        mesh=_sc_mesh(),
        compiler_params=pltpu.CompilerParams(use_tc_tiling_on_sc=False),
        scratch_types=[
            pltpu.VMEM((NCH, CHUNK), jnp.int32),
            pltpu.VMEM((CHUNK,), jnp.float32),
            pltpu.VMEM((NDEG // NS,), jnp.float32),
            pltpu.VMEM_SHARED((NDEG,), jnp.float32),
        ],
    )(dstidx)


def _prop_body(C, g2, srcidx, dstidx, out, src_v, dst_v, buf, zbuf, acc, sem):
    core = lax.axis_index("c")
    sub = lax.axis_index("s")
    wid = sub * NC + core
    pltpu.sync_copy(srcidx.at[wid], src_v)
    pltpu.sync_copy(dstidx.at[wid], dst_v)

    GPR = F // 16

    def _zb(i, _):
        zbuf[i // GPR, pl.ds((i % GPR) * 16, 16)] = jnp.zeros((16,), jnp.float32)
        return _

    lax.fori_loop(0, 320 * GPR, _zb, None)

    for cc in range(C):
        pltpu.sync_copy(zbuf, acc.at[pl.ds(sub * STRIPE, 320)])
        pltpu.sync_copy(zbuf, acc.at[pl.ds(sub * STRIPE + 320, 320)])
        plsc.subcore_barrier()

        def _edge(j, _):
            pltpu.async_copy(g2.at[src_v.at[cc, j]], buf, sem).wait()
            pltpu.sync_copy(buf, acc.at[dst_v.at[j]], add=True)
            return _

        lax.fori_loop(0, NCH, _edge, None)
        plsc.subcore_barrier()
        for k in range(NC):
            @pl.when(core == k)
            def _(k=k, cc=cc):
                pltpu.sync_copy(
                    acc.at[pl.ds(sub * STRIPE, STRIPE)],
                    out.at[k, pl.ds(cc * NPAD + sub * STRIPE, STRIPE)])
        plsc.subcore_barrier()


def _prop(C, g2, srcidx, dstidx):
    return pl.kernel(
        functools.partial(_prop_body, C),
        out_type=jax.ShapeDtypeStruct((NC, C * NPAD, F), jnp.float32),
        mesh=_sc_mesh(),
        compiler_params=pltpu.CompilerParams(use_tc_tiling_on_sc=False),
        scratch_types=[
            pltpu.VMEM((C, NCH, CHUNK), jnp.int32),
            pltpu.VMEM((NCH, CHUNK), jnp.int32),
            pltpu.VMEM((CHUNK, F), jnp.float32),
            pltpu.VMEM((320, F), jnp.float32),
            pltpu.VMEM_SHARED((NPAD, F), jnp.float32),
            pltpu.SemaphoreType.DMA,
        ],
    )(g2, srcidx, dstidx)


def _ta_body(degp_ref, x_ref, g_ref, norm_ref):
    deg = degp_ref[0] + degp_ref[1]
    nrm = lax.rsqrt(jnp.maximum(deg, 1.0))
    norm_ref[...] = nrm
    s = x_ref[...] * nrm
    for c in range(IN_FEATS // F):
        g_ref[c] = s[:, c * F:(c + 1) * F]


def _tc_prescale(degp, features):
    CI = IN_FEATS // F
    return pl.pallas_call(
        _ta_body,
        grid=(N // BN,),
        in_specs=[
            pl.BlockSpec((NC, BN, 1), lambda i: (0, i, 0)),
            pl.BlockSpec((BN, IN_FEATS), lambda i: (i, 0)),
        ],
        out_specs=[
            pl.BlockSpec((CI, BN, F), lambda i: (0, i, 0)),
            pl.BlockSpec((BN, 1), lambda i: (i, 0)),
        ],
        out_shape=[
            jax.ShapeDtypeStruct((CI, N, F), jnp.float32),
            jax.ShapeDtypeStruct((N, 1), jnp.float32),
        ],
    )(degp[:, :N, None], features)


def _tb_body(p_ref, norm_ref, m_ref):
    nrm = norm_ref[...]
    m_ref[0] = (p_ref[0, 0] + p_ref[1, 0]) * (nrm * nrm)


def _tc_mid(p, norm, C):
    p4 = p.reshape(NC, C, NPAD, F)
    return pl.pallas_call(
        _tb_body,
        grid=(C, N // BN),
        in_specs=[
            pl.BlockSpec((NC, 1, BN, F), lambda c, i: (0, c, i, 0)),
            pl.BlockSpec((BN, 1), lambda c, i: (i, 0)),
        ],
        out_specs=pl.BlockSpec((1, BN, F), lambda c, i: (c, i, 0)),
        out_shape=jax.ShapeDtypeStruct((C, N, F), jnp.float32),
    )(p4, norm)


def _tc_layer_body(CI, CO, p_ref, norm_ref, w_ref, g_ref):
    nrm = norm_ref[...]
    acc = jnp.zeros((BN, w_ref.shape[0]), jnp.float32)
    for c in range(CI):
        t = (p_ref[0, c] + p_ref[1, c]) * nrm
        acc = acc + lax.dot_general(
            t, w_ref[:, c * F:(c + 1) * F],
            (((1,), (1,)), ((), ())), preferred_element_type=jnp.float32)
    h = jnp.maximum(acc, 0.0) * nrm
    for co in range(CO):
        g_ref[co] = h[:, co * F:(co + 1) * F]


def _tc_layer(p, norm, W, CI, CO):
    p4 = p.reshape(NC, CI, NPAD, F)
    return pl.pallas_call(
        functools.partial(_tc_layer_body, CI, CO),
        grid=(N // BN,),
        in_specs=[
            pl.BlockSpec((NC, CI, BN, F), lambda i: (0, 0, i, 0)),
            pl.BlockSpec((BN, 1), lambda i: (i, 0)),
            pl.BlockSpec(W.shape, lambda i: (0, 0)),
        ],
        out_specs=pl.BlockSpec((CO, BN, F), lambda i: (0, i, 0)),
        out_shape=jax.ShapeDtypeStruct((CO, N, F), jnp.float32),
    )(p4, norm, W)


def _tc_layer2_body(CI, CO, p_ref, norm_ref, w2_ref, w3_ref, g_ref):
    nrm = norm_ref[...]
    acc = jnp.zeros((BN, N_HIDDEN), jnp.float32)
    for c in range(CI):
        t = (p_ref[0, c] + p_ref[1, c]) * nrm
        acc = acc + lax.dot_general(
            t, w2_ref[:, c * F:(c + 1) * F],
            (((1,), (1,)), ((), ())), preferred_element_type=jnp.float32)
    h = jnp.maximum(acc, 0.0)
    z = lax.dot_general(h, w3_ref[...], (((1,), (1,)), ((), ())),
                        preferred_element_type=jnp.float32)
    g = z * nrm
    for co in range(CO):
        g_ref[co] = g[:, co * F:(co + 1) * F]


def _tc_layer2(p, norm, W2, W3, CI, CO):
    p4 = p.reshape(NC, CI, NPAD, F)
    return pl.pallas_call(
        functools.partial(_tc_layer2_body, CI, CO),
        grid=(N // BN,),
        in_specs=[
            pl.BlockSpec((NC, CI, BN, F), lambda i: (0, 0, i, 0)),
            pl.BlockSpec((BN, 1), lambda i: (i, 0)),
            pl.BlockSpec(W2.shape, lambda i: (0, 0)),
            pl.BlockSpec(W3.shape, lambda i: (0, 0)),
        ],
        out_specs=pl.BlockSpec((CO, BN, F), lambda i: (0, i, 0)),
        out_shape=jax.ShapeDtypeStruct((CO, N, F), jnp.float32),
    )(p4, norm, W2, W3)


def _td_body(p_ref, norm_ref, o_ref):
    nrm = norm_ref[...]
    cols = [(p_ref[0, c] + p_ref[1, c]) * nrm for c in range(N_CLASSES // F)]
    o_ref[...] = jnp.concatenate(cols, axis=1)


def _tc_final(p, norm):
    CI = N_CLASSES // F
    p4 = p.reshape(NC, CI, NPAD, F)
    return pl.pallas_call(
        _td_body,
        grid=(N // BN,),
        in_specs=[
            pl.BlockSpec((NC, CI, BN, F), lambda i: (0, 0, i, 0)),
            pl.BlockSpec((BN, 1), lambda i: (i, 0)),
        ],
        out_specs=pl.BlockSpec((BN, N_CLASSES), lambda i: (i, 0)),
        out_shape=jax.ShapeDtypeStruct((N, N_CLASSES), jnp.float32),
    )(p4, norm)


def kernel(features, edge_index, W1, W2, W3):
    src = edge_index[0]
    dst = edge_index[1]

    w = jnp.arange(NW, dtype=jnp.int32)[:, None]
    i = jnp.arange(PADE, dtype=jnp.int32)[None, :]
    pad_src = (w * 997 + i * 131) % N
    pad_dst = N + (i % 8) + jnp.zeros((NW, 1), jnp.int32)
    srcp = jnp.concatenate([src.reshape(NW, EPW), pad_src], axis=1)
    dstp = jnp.concatenate([dst.reshape(NW, EPW), pad_dst], axis=1)
    dsti = dstp.reshape(NW, NCH, CHUNK)

    def srci(C):
        off = jnp.arange(C, dtype=jnp.int32)[None, :, None] * N
        return (srcp[:, None, :] + off).reshape(NW, C, NCH, CHUNK)

    srci4, srci8 = srci(IN_FEATS // F), srci(N_HIDDEN // F)

    degp = _deg_kernel(dsti)

    CA = IN_FEATS // F
    CB = N_HIDDEN // F
    g, norm = _tc_prescale(degp, features)
    p = _prop(CA, g.reshape(CA * N, F), srci4, dsti)
    m = _tc_mid(p, norm, CA)
    p = _prop(CA, m.reshape(CA * N, F), srci4, dsti)
    g = _tc_layer(p, norm, W1, CA, CB)
    p = _prop(CB, g.reshape(CB * N, F), srci8, dsti)
    m = _tc_mid(p, norm, CB)
    p = _prop(CB, m.reshape(CB * N, F), srci8, dsti)
    g = _tc_layer2(p, norm, W2, W3, CB, CA)
    p = _prop(CA, g.reshape(CA * N, F), srci4, dsti)
    m = _tc_mid(p, norm, CA)
    p = _prop(CA, m.reshape(CA * N, F), srci4, dsti)
    return _tc_final(p, norm)


# speedup vs baseline: 4.5180x; 1.3197x over previous
import functools

import jax
import jax.numpy as jnp
from jax import lax
from jax.experimental import pallas as pl
from jax.experimental.pallas import tpu as pltpu
from jax.experimental.pallas import tpu_sc as plsc

N = 10000
E = 160000
IN_FEATS = 256
N_HIDDEN = 512
N_CLASSES = 256

NC = 2
NS = 16
NW = NC * NS
EPW = E // NW
CHUNK = 128
NCH = -(-EPW // CHUNK)
EPW_PAD = NCH * CHUNK
PADE = EPW_PAD - EPW
NPAD = 10240
STRIPE = NPAD // NS
F = 64
BN = 1000


def _sc_mesh():
    return plsc.VectorSubcoreMesh(core_axis_name="c", subcore_axis_name="s")


NDEG = 10240


def _deg_body(dstidx, degp, dst_v, ones_v, zbuf, acc):
    core = lax.axis_index("c")
    sub = lax.axis_index("s")
    wid = sub * NC + core
    pltpu.sync_copy(dstidx.at[wid], dst_v)

    def _fill(i, _):
        ones_v[pl.ds(i * 16, 16)] = jnp.full((16,), 1.0, jnp.float32)
        return _

    def _zero(i, _):
        zbuf[pl.ds(i * 16, 16)] = jnp.zeros((16,), jnp.float32)
        return _

    lax.fori_loop(0, CHUNK // 16, _fill, None)
    lax.fori_loop(0, (NDEG // NS) // 16, _zero, None)
    pltpu.sync_copy(zbuf, acc.at[pl.ds(sub * (NDEG // NS), NDEG // NS)])
    plsc.subcore_barrier()

    def _scat(j, _):
        pltpu.sync_copy(ones_v, acc.at[dst_v.at[j]], add=True)
        return _

    lax.fori_loop(0, NCH, _scat, None)
    plsc.subcore_barrier()
    for k in range(NC):
        @pl.when(core == k)
        def _(k=k):
            pltpu.sync_copy(acc.at[pl.ds(sub * (NDEG // NS), NDEG // NS)],
                            degp.at[k, pl.ds(sub * (NDEG // NS), NDEG // NS)])


def _deg_kernel(dstidx):
    return pl.kernel(
        _deg_body,
        out_type=jax.ShapeDtypeStruct((NC, NDEG), jnp.float32),
        mesh=_sc_mesh(),
        compiler_params=pltpu.CompilerParams(use_tc_tiling_on_sc=False),
        scratch_types=[
            pltpu.VMEM((NCH, CHUNK), jnp.int32),
            pltpu.VMEM((CHUNK,), jnp.float32),
            pltpu.VMEM((NDEG // NS,), jnp.float32),
            pltpu.VMEM_SHARED((NDEG,), jnp.float32),
        ],
    )(dstidx)


NBUF = 4


def _prop_body(C, g2, srcidx, dstidx, out, src_v, dst_v, bufs, zbuf, acc,
               gsem, ssem):
    core = lax.axis_index("c")
    sub = lax.axis_index("s")
    wid = sub * NC + core
    pltpu.sync_copy(srcidx.at[wid], src_v)
    pltpu.sync_copy(dstidx.at[wid], dst_v)

    GPR = F // 16

    def _zb(i, _):
        zbuf[i // GPR, pl.ds((i % GPR) * 16, 16)] = jnp.zeros((16,), jnp.float32)
        return _

    lax.fori_loop(0, 160 * GPR, _zb, None)

    for cc in range(C):
        for q in range(4):
            pltpu.sync_copy(zbuf, acc.at[pl.ds(sub * STRIPE + q * 160, 160)])
        plsc.subcore_barrier()

        def _grp(t, _):
            j0 = t * NBUF
            gds = [pltpu.async_copy(g2.at[src_v.at[cc, j0 + b]],
                                    bufs.at[b], gsem.at[b])
                   for b in range(NBUF)]
            sds = []
            for b in range(NBUF):
                gds[b].wait()
                sds.append(pltpu.async_copy(bufs.at[b],
                                            acc.at[dst_v.at[j0 + b]],
                                            ssem.at[b], add=True))
            for sd in sds:
                sd.wait()
            return _

        lax.fori_loop(0, NCH // NBUF, _grp, None)
        plsc.subcore_barrier()
        for k in range(NC):
            @pl.when(core == k)
            def _(k=k, cc=cc):
                pltpu.sync_copy(
                    acc.at[pl.ds(sub * STRIPE, STRIPE)],
                    out.at[k, pl.ds(cc * NPAD + sub * STRIPE, STRIPE)])
        plsc.subcore_barrier()


def _prop(C, g2, srcidx, dstidx):
    return pl.kernel(
        functools.partial(_prop_body, C),
        out_type=jax.ShapeDtypeStruct((NC, C * NPAD, F), jnp.float32),
        mesh=_sc_mesh(),
        compiler_params=pltpu.CompilerParams(use_tc_tiling_on_sc=False),
        scratch_types=[
            pltpu.VMEM((C, NCH, CHUNK), jnp.int32),
            pltpu.VMEM((NCH, CHUNK), jnp.int32),
            pltpu.VMEM((NBUF, CHUNK, F), jnp.float32),
            pltpu.VMEM((160, F), jnp.float32),
            pltpu.VMEM_SHARED((NPAD, F), jnp.float32),
            pltpu.SemaphoreType.DMA((NBUF,)),
            pltpu.SemaphoreType.DMA((NBUF,)),
        ],
    )(g2, srcidx, dstidx)


def _ta_body(degp_ref, x_ref, g_ref, norm_ref):
    deg = degp_ref[0] + degp_ref[1]
    nrm = lax.rsqrt(jnp.maximum(deg, 1.0))
    norm_ref[...] = nrm
    s = x_ref[...] * nrm
    for c in range(IN_FEATS // F):
        g_ref[c] = s[:, c * F:(c + 1) * F]


def _tc_prescale(degp, features):
    CI = IN_FEATS // F
    return pl.pallas_call(
        _ta_body,
        grid=(N // BN,),
        in_specs=[
            pl.BlockSpec((NC, BN, 1), lambda i: (0, i, 0)),
            pl.BlockSpec((BN, IN_FEATS), lambda i: (i, 0)),
        ],
        out_specs=[
            pl.BlockSpec((CI, BN, F), lambda i: (0, i, 0)),
            pl.BlockSpec((BN, 1), lambda i: (i, 0)),
        ],
        out_shape=[
            jax.ShapeDtypeStruct((CI, N, F), jnp.float32),
            jax.ShapeDtypeStruct((N, 1), jnp.float32),
        ],
    )(degp[:, :N, None], features)


def _tb_body(p_ref, norm_ref, m_ref):
    nrm = norm_ref[...]
    m_ref[0] = (p_ref[0, 0] + p_ref[1, 0]) * (nrm * nrm)


def _tc_mid(p, norm, C):
    p4 = p.reshape(NC, C, NPAD, F)
    return pl.pallas_call(
        _tb_body,
        grid=(C, N // BN),
        in_specs=[
            pl.BlockSpec((NC, 1, BN, F), lambda c, i: (0, c, i, 0)),
            pl.BlockSpec((BN, 1), lambda c, i: (i, 0)),
        ],
        out_specs=pl.BlockSpec((1, BN, F), lambda c, i: (c, i, 0)),
        out_shape=jax.ShapeDtypeStruct((C, N, F), jnp.float32),
    )(p4, norm)


def _tc_layer_body(CI, CO, p_ref, norm_ref, w_ref, g_ref):
    nrm = norm_ref[...]
    acc = jnp.zeros((BN, w_ref.shape[0]), jnp.float32)
    for c in range(CI):
        t = (p_ref[0, c] + p_ref[1, c]) * nrm
        acc = acc + lax.dot_general(
            t, w_ref[:, c * F:(c + 1) * F],
            (((1,), (1,)), ((), ())), preferred_element_type=jnp.float32)
    h = jnp.maximum(acc, 0.0) * nrm
    for co in range(CO):
        g_ref[co] = h[:, co * F:(co + 1) * F]


def _tc_layer(p, norm, W, CI, CO):
    p4 = p.reshape(NC, CI, NPAD, F)
    return pl.pallas_call(
        functools.partial(_tc_layer_body, CI, CO),
        grid=(N // BN,),
        in_specs=[
            pl.BlockSpec((NC, CI, BN, F), lambda i: (0, 0, i, 0)),
            pl.BlockSpec((BN, 1), lambda i: (i, 0)),
            pl.BlockSpec(W.shape, lambda i: (0, 0)),
        ],
        out_specs=pl.BlockSpec((CO, BN, F), lambda i: (0, i, 0)),
        out_shape=jax.ShapeDtypeStruct((CO, N, F), jnp.float32),
    )(p4, norm, W)


def _tc_layer2_body(CI, CO, p_ref, norm_ref, w2_ref, w3_ref, g_ref):
    nrm = norm_ref[...]
    acc = jnp.zeros((BN, N_HIDDEN), jnp.float32)
    for c in range(CI):
        t = (p_ref[0, c] + p_ref[1, c]) * nrm
        acc = acc + lax.dot_general(
            t, w2_ref[:, c * F:(c + 1) * F],
            (((1,), (1,)), ((), ())), preferred_element_type=jnp.float32)
    h = jnp.maximum(acc, 0.0)
    z = lax.dot_general(h, w3_ref[...], (((1,), (1,)), ((), ())),
                        preferred_element_type=jnp.float32)
    g = z * nrm
    for co in range(CO):
        g_ref[co] = g[:, co * F:(co + 1) * F]


def _tc_layer2(p, norm, W2, W3, CI, CO):
    p4 = p.reshape(NC, CI, NPAD, F)
    return pl.pallas_call(
        functools.partial(_tc_layer2_body, CI, CO),
        grid=(N // BN,),
        in_specs=[
            pl.BlockSpec((NC, CI, BN, F), lambda i: (0, 0, i, 0)),
            pl.BlockSpec((BN, 1), lambda i: (i, 0)),
            pl.BlockSpec(W2.shape, lambda i: (0, 0)),
            pl.BlockSpec(W3.shape, lambda i: (0, 0)),
        ],
        out_specs=pl.BlockSpec((CO, BN, F), lambda i: (0, i, 0)),
        out_shape=jax.ShapeDtypeStruct((CO, N, F), jnp.float32),
    )(p4, norm, W2, W3)


def _td_body(p_ref, norm_ref, o_ref):
    nrm = norm_ref[...]
    cols = [(p_ref[0, c] + p_ref[1, c]) * nrm for c in range(N_CLASSES // F)]
    o_ref[...] = jnp.concatenate(cols, axis=1)


def _tc_final(p, norm):
    CI = N_CLASSES // F
    p4 = p.reshape(NC, CI, NPAD, F)
    return pl.pallas_call(
        _td_body,
        grid=(N // BN,),
        in_specs=[
            pl.BlockSpec((NC, CI, BN, F), lambda i: (0, 0, i, 0)),
            pl.BlockSpec((BN, 1), lambda i: (i, 0)),
        ],
        out_specs=pl.BlockSpec((BN, N_CLASSES), lambda i: (i, 0)),
        out_shape=jax.ShapeDtypeStruct((N, N_CLASSES), jnp.float32),
    )(p4, norm)


def kernel(features, edge_index, W1, W2, W3):
    src = edge_index[0]
    dst = edge_index[1]

    w = jnp.arange(NW, dtype=jnp.int32)[:, None]
    i = jnp.arange(PADE, dtype=jnp.int32)[None, :]
    pad_src = (w * 997 + i * 131) % N
    pad_dst = N + (i % 8) + jnp.zeros((NW, 1), jnp.int32)
    srcp = jnp.concatenate([src.reshape(NW, EPW), pad_src], axis=1)
    dstp = jnp.concatenate([dst.reshape(NW, EPW), pad_dst], axis=1)
    dsti = dstp.reshape(NW, NCH, CHUNK)

    def srci(C):
        off = jnp.arange(C, dtype=jnp.int32)[None, :, None] * N
        return (srcp[:, None, :] + off).reshape(NW, C, NCH, CHUNK)

    srci4, srci8 = srci(IN_FEATS // F), srci(N_HIDDEN // F)

    degp = _deg_kernel(dsti)

    CA = IN_FEATS // F
    CB = N_HIDDEN // F
    g, norm = _tc_prescale(degp, features)
    p = _prop(CA, g.reshape(CA * N, F), srci4, dsti)
    m = _tc_mid(p, norm, CA)
    p = _prop(CA, m.reshape(CA * N, F), srci4, dsti)
    g = _tc_layer(p, norm, W1, CA, CB)
    p = _prop(CB, g.reshape(CB * N, F), srci8, dsti)
    m = _tc_mid(p, norm, CB)
    p = _prop(CB, m.reshape(CB * N, F), srci8, dsti)
    g = _tc_layer2(p, norm, W2, W3, CB, CA)
    p = _prop(CA, g.reshape(CA * N, F), srci4, dsti)
    m = _tc_mid(p, norm, CA)
    p = _prop(CA, m.reshape(CA * N, F), srci4, dsti)
    return _tc_final(p, norm)
